# baseline (speedup 1.0000x reference)
import functools

import jax
import jax.numpy as jnp
from jax import lax
from jax.experimental import pallas as pl
from jax.experimental.pallas import tpu as pltpu
from jax.experimental.pallas import tpu_sc as plsc

BATCH = 16384
HIST = 50
EMBED = 32
N = BATCH * HIST
NUM_WORKERS = 32
PER_WORKER = N // NUM_WORKERS
CHUNK = 1280
HALF = CHUNK // 2
NUM_CHUNKS = PER_WORKER // CHUNK
HALF_STEPS = NUM_CHUNKS // 2

_mesh = plsc.VectorSubcoreMesh(core_axis_name="c", subcore_axis_name="s")


@functools.partial(
    pl.kernel,
    mesh=_mesh,
    out_type=jax.ShapeDtypeStruct((N, EMBED), jnp.float32),
    scratch_types=[
        pltpu.VMEM((PER_WORKER,), jnp.int32),
        pltpu.VMEM((CHUNK, EMBED), jnp.float32),
        pltpu.VMEM((CHUNK, EMBED), jnp.float32),
        pltpu.SemaphoreType.DMA,
        pltpu.SemaphoreType.DMA,
        pltpu.SemaphoreType.DMA,
        pltpu.SemaphoreType.DMA,
        pltpu.SemaphoreType.DMA,
        pltpu.SemaphoreType.DMA,
    ],
    compiler_params=pltpu.CompilerParams(use_tc_tiling_on_sc=False),
)
def _gather_kernel(idx_hbm, table_hbm, out_hbm, idx_v, rows0, rows1,
                   ga0, gb0, ga1, gb1, osem0, osem1):
    gsems = [(ga0, gb0), (ga1, gb1)]
    osems = [osem0, osem1]
    bufs = [rows0, rows1]

    wid = lax.axis_index("s") * 2 + lax.axis_index("c")
    base = wid * PER_WORKER

    pltpu.sync_copy(idx_hbm.at[pl.ds(base, PER_WORKER)], idx_v)

    def half_slice(g, h):
        return idx_v.at[pl.ds(pl.multiple_of(g * CHUNK + h * HALF, 8), HALF)]

    def fire_gather(g, j):
        buf = bufs[j]
        sa, sb = gsems[j]
        pltpu.async_copy(table_hbm.at[half_slice(g, 0)], buf.at[pl.ds(0, HALF)], sa)
        pltpu.async_copy(table_hbm.at[half_slice(g, 1)], buf.at[pl.ds(HALF, HALF)], sb)

    def wait_gather(j):
        buf = bufs[j]
        sa, sb = gsems[j]
        pltpu.make_async_copy(
            table_hbm.at[half_slice(0, 0)], buf.at[pl.ds(0, HALF)], sa).wait()
        pltpu.make_async_copy(
            table_hbm.at[half_slice(0, 1)], buf.at[pl.ds(HALF, HALF)], sb).wait()

    def out_ref(g):
        return out_hbm.at[pl.ds(base + pl.multiple_of(g * CHUNK, 8), CHUNK)]

    def fire_out(g, j):
        pltpu.async_copy(bufs[j], out_ref(g), osems[j])

    def wait_out(g, j):
        pltpu.make_async_copy(bufs[j], out_ref(g), osems[j]).wait()

    fire_gather(0, 0)

    def body(i, carry):
        g0 = i * 2
        wait_gather(0)

        @pl.when(i > 0)
        def _():
            wait_out(g0 - 1, 1)

        fire_gather(g0 + 1, 1)
        fire_out(g0, 0)

        wait_gather(1)
        wait_out(g0, 0)

        @pl.when(i < HALF_STEPS - 1)
        def _():
            fire_gather(g0 + 2, 0)

        fire_out(g0 + 1, 1)
        return carry

    lax.fori_loop(0, HALF_STEPS, body, 0)
    wait_out(NUM_CHUNKS - 1, 1)


def kernel(indices, table):
    idx = indices.reshape(-1).astype(jnp.int32)
    out = _gather_kernel(idx, table)
    return out.reshape(BATCH, HIST, EMBED)
